# baseline (speedup 1.0000x reference)
import functools

import jax
import jax.numpy as jnp
from jax import lax
from jax.experimental import pallas as pl
from jax.experimental.pallas import tpu as pltpu
from jax.experimental.pallas import tpu_sc as plsc

_USER = 5000
_N = 10000
_NPAD = 10112
_EMB = 128
_E = 320000
_BATCH = 4096
_NC, _NS = 2, 16
_NW = _NC * _NS
_K = 80
_CH = 126
_EPT = _CH * _K
_EPAD = _EPT * _NW
_RPT = _NPAD // _NS

_mesh = plsc.VectorSubcoreMesh(core_axis_name="c", subcore_axis_name="s")


def _bvec(v16, j):
    return jnp.take(v16, jnp.full((16,), j, dtype=jnp.int32))


@functools.partial(
    pl.kernel,
    out_type=jax.ShapeDtypeStruct((_NC, _NPAD, _EMB), jnp.float32),
    mesh=_mesh,
    scratch_types=dict(
        cb0=pltpu.VMEM((_K,), jnp.int32),
        cb1=pltpu.VMEM((_K,), jnp.int32),
        cb2=pltpu.VMEM((_K,), jnp.int32),
        vb0=pltpu.VMEM((_K,), jnp.float32),
        vb1=pltpu.VMEM((_K,), jnp.float32),
        vb2=pltpu.VMEM((_K,), jnp.float32),
        rb0=pltpu.VMEM((_K,), jnp.int32),
        rb1=pltpu.VMEM((_K,), jnp.int32),
        rb2=pltpu.VMEM((_K,), jnp.int32),
        gb0=pltpu.VMEM((_K, _EMB), jnp.float32),
        gb1=pltpu.VMEM((_K, _EMB), jnp.float32),
        gb2=pltpu.VMEM((_K, _EMB), jnp.float32),
        semc0=pltpu.SemaphoreType.DMA, semc1=pltpu.SemaphoreType.DMA,
        semc2=pltpu.SemaphoreType.DMA, semg0=pltpu.SemaphoreType.DMA,
        semg1=pltpu.SemaphoreType.DMA, semg2=pltpu.SemaphoreType.DMA,
        sems0=pltpu.SemaphoreType.DMA, sems1=pltpu.SemaphoreType.DMA,
        sems2=pltpu.SemaphoreType.DMA, semr0=pltpu.SemaphoreType.DMA,
        semr1=pltpu.SemaphoreType.DMA, semr2=pltpu.SemaphoreType.DMA,
        acc=pltpu.VMEM_SHARED((_NPAD, _EMB), jnp.float32),
    ),
)
def _layer(ego, colsr, valsr, rowsr, part, *, cb0, cb1, cb2, vb0, vb1, vb2,
           rb0, rb1, rb2, gb0, gb1, gb2, semc0, semc1, semc2, semg0, semg1,
           semg2, sems0, sems1, sems2, semr0, semr1, semr2, acc):
    cid = lax.axis_index("c")
    sid = lax.axis_index("s")
    wid = cid * _NS + sid
    cb = (cb0, cb1, cb2)
    vb = (vb0, vb1, vb2)
    rb = (rb0, rb1, rb2)
    gb = (gb0, gb1, gb2)
    semc = (semc0, semc1, semc2)
    semg = (semg0, semg1, semg2)
    sems = (sems0, sems1, sems2)
    semr = (semr0, semr1, semr2)
    cbase = wid * _CH

    z16 = jnp.zeros((16,), jnp.float32)

    def _zero_row(i, carry):
        for cc in range(_EMB // 16):
            gb0[i, pl.ds(cc * 16, 16)] = z16
        return carry

    lax.fori_loop(0, _K, _zero_row, 0)
    rstart = sid * _RPT
    for z in range(_RPT // _K):
        pltpu.sync_copy(gb0, acc.at[pl.ds(rstart + z * _K, _K)])
    pltpu.sync_copy(gb0.at[pl.ds(0, _RPT % _K)],
                    acc.at[pl.ds(rstart + (_RPT // _K) * _K, _RPT % _K)])

    plsc.subcore_barrier()

    def _meta_issue(t, r):
        pltpu.async_copy(colsr.at[cbase + t], cb[r], semc[r])
        pltpu.async_copy(valsr.at[cbase + t], vb[r], semc[r])

    def _meta_wait(t, r):
        pltpu.make_async_copy(colsr.at[cbase + t], cb[r], semc[r]).wait()
        pltpu.make_async_copy(valsr.at[cbase + t], vb[r], semc[r]).wait()

    def _rows_issue(t, r):
        pltpu.async_copy(rowsr.at[cbase + t], rb[r], semr[r])

    def _rows_wait(t, r):
        pltpu.make_async_copy(rowsr.at[cbase + t], rb[r], semr[r]).wait()

    def _gather_issue(t, r):
        pltpu.async_copy(ego.at[cb[r]], gb[r], semg[r])

    def _gather_wait(r):
        pltpu.make_async_copy(ego.at[cb[r]], gb[r], semg[r]).wait()

    def _scatter_issue(t, r):
        pltpu.async_copy(gb[r], acc.at[rb[r]], sems[r], add=True)

    def _scatter_wait(r):
        pltpu.make_async_copy(gb[r], acc.at[rb[r]], sems[r]).wait()

    def _compute(r):
        gbr = gb[r]
        vbr = vb[r]

        def _grp(g, carry):
            v16 = vbr[pl.ds(g * 16, 16)]
            for j in range(16):
                e = g * 16 + j
                bv = _bvec(v16, j)
                for cc in range(_EMB // 16):
                    gbr[e, pl.ds(cc * 16, 16)] = (
                        gbr[e, pl.ds(cc * 16, 16)] * bv)
            return carry

        lax.fori_loop(0, _K // 16, _grp, 0)

    def _slot(t, r, do_next, do_meta, do_free):
        rn = (r + 1) % 3
        if do_next:
            _meta_wait(t + 1, rn)
            if do_free:
                _scatter_wait(rn)
            _rows_issue(t + 1, rn)
            _gather_issue(t + 1, rn)
        _gather_wait(r)
        _compute(r)
        _rows_wait(t, r)
        _scatter_issue(t, r)
        if do_meta:
            _meta_issue(t + 2, (r + 2) % 3)

    _meta_issue(0, 0)
    _meta_issue(1, 1)
    _rows_issue(0, 0)
    _meta_wait(0, 0)
    _gather_issue(0, 0)

    _slot(0, 0, True, True, False)
    _slot(1, 1, True, True, False)
    _slot(2, 2, True, True, True)

    def _main(q, carry):
        t = 3 + q * 3
        _slot(t, 0, True, True, True)
        _slot(t + 1, 1, True, True, True)
        _slot(t + 2, 2, True, True, True)
        return carry

    lax.fori_loop(0, (_CH - 6) // 3, _main, 0)

    _slot(_CH - 3, 0, True, True, True)
    _slot(_CH - 2, 1, True, False, True)
    _slot(_CH - 1, 2, False, False, False)

    _scatter_wait(0)
    _scatter_wait(1)
    _scatter_wait(2)

    plsc.subcore_barrier()

    pltpu.sync_copy(acc.at[pl.ds(rstart, _RPT)],
                    part.at[cid, pl.ds(rstart, _RPT)])


_CBLK = 632


def _combine_body(p_ref, o_ref):
    o_ref[...] = p_ref[0] + p_ref[1]


def _combine(part):
    return pl.pallas_call(
        _combine_body,
        out_shape=jax.ShapeDtypeStruct((_NPAD, _EMB), jnp.float32),
        grid=(_NPAD // _CBLK,),
        in_specs=[pl.BlockSpec((_NC, _CBLK, _EMB), lambda i: (0, i, 0))],
        out_specs=pl.BlockSpec((_CBLK, _EMB), lambda i: (i, 0)),
    )(part)


_B2 = 2 * _BATCH
_GK = 32
_GCH = _B2 // _GK // _NW


@functools.partial(
    pl.kernel,
    out_type=jax.ShapeDtypeStruct((_B2, _EMB), jnp.float32),
    mesh=_mesh,
    scratch_types=dict(
        ibuf=pltpu.VMEM((_GCH, _GK), jnp.int32),
        gbufa=pltpu.VMEM((_GK, _EMB), jnp.float32),
        gbufb=pltpu.VMEM((_GK, _EMB), jnp.float32),
        obuf=pltpu.VMEM((_GK, _EMB), jnp.float32),
        sema=pltpu.SemaphoreType.DMA,
        semb=pltpu.SemaphoreType.DMA,
    ),
)
def _gather_mean(ego0, ego1, ego2, p3a, p3b, bidx, out, *, ibuf, gbufa,
                 gbufb, obuf, sema, semb):
    cid = lax.axis_index("c")
    sid = lax.axis_index("s")
    wid = cid * _NS + sid
    pltpu.sync_copy(bidx.at[wid], ibuf)
    quarter = jnp.full((16,), 0.25, dtype=jnp.float32)
    srcs = (ego0, ego1, ego2, p3a, p3b)
    nsrc = len(srcs)
    gb = (gbufa, gbufb)
    sem = (sema, semb)

    def _issue(k):
        t, s = divmod(k, nsrc)
        pltpu.async_copy(srcs[s].at[ibuf.at[t]], gb[k % 2], sem[k % 2])

    def _wait(k):
        t, s = divmod(k, nsrc)
        pltpu.make_async_copy(srcs[s].at[ibuf.at[t]], gb[k % 2],
                              sem[k % 2]).wait()

    def _acc_rows(k):
        g = gb[k % 2]
        first = k % nsrc == 0

        def _row(i, c2):
            for cc in range(_EMB // 16):
                v = g[i, pl.ds(cc * 16, 16)]
                if first:
                    obuf[i, pl.ds(cc * 16, 16)] = v
                else:
                    obuf[i, pl.ds(cc * 16, 16)] = (
                        obuf[i, pl.ds(cc * 16, 16)] + v)
            return c2

        lax.fori_loop(0, _GK, _row, 0)

    def _flush(t):
        def _scale(i, c2):
            for cc in range(_EMB // 16):
                obuf[i, pl.ds(cc * 16, 16)] = (
                    obuf[i, pl.ds(cc * 16, 16)] * quarter)
            return c2

        lax.fori_loop(0, _GK, _scale, 0)
        pltpu.sync_copy(obuf, out.at[pl.ds((wid * _GCH + t) * _GK, _GK)])

    total = _GCH * nsrc
    _issue(0)
    for k in range(total):
        if k + 1 < total:
            _issue(k + 1)
        _wait(k)
        _acc_rows(k)
        if k % nsrc == nsrc - 1:
            _flush(k // nsrc)


def kernel(users, items, user_emb, item_emb, adj_rows, adj_cols, adj_vals):
    ego0 = jnp.concatenate([user_emb, item_emb], axis=0)

    npad = _EPAD - _E
    cols_p = jnp.concatenate([adj_cols, jnp.zeros((npad,), jnp.int32)])
    rows_p = jnp.concatenate([adj_rows, jnp.zeros((npad,), jnp.int32)])
    vals_p = jnp.concatenate([adj_vals, jnp.zeros((npad,), jnp.float32)])
    colsr = cols_p.reshape(_NW * _CH, _K)
    valsr = vals_p.reshape(_NW * _CH, _K)
    rowsr = rows_p.reshape(_NW * _CH, _K)

    p1 = _layer(ego0, colsr, valsr, rowsr)
    ego1 = _combine(p1)
    p2 = _layer(ego1, colsr, valsr, rowsr)
    ego2 = _combine(p2)
    p3 = _layer(ego2, colsr, valsr, rowsr)

    bidx = jnp.concatenate([users, items + _USER]).reshape(_NW, _GCH, _GK)
    out = _gather_mean(ego0, ego1, ego2, p3[0], p3[1], bidx)
    return out[:_BATCH], out[_BATCH:]


# speedup vs baseline: 1.1116x; 1.1116x over previous
import functools

import jax
import jax.numpy as jnp
from jax import lax
from jax.experimental import pallas as pl
from jax.experimental.pallas import tpu as pltpu
from jax.experimental.pallas import tpu_sc as plsc

_USER = 5000
_N = 10000
_NPAD = 10112
_EMB = 128
_E = 320000
_BATCH = 4096
_NC, _NS = 2, 16
_NW = _NC * _NS
_K = 96
_CH = 105
_CH0 = 120
_CH1 = 2 * _CH - _CH0
_EPT = _CH * _K
_EPAD = _EPT * _NW
_RPT = _NPAD // _NS

_mesh = plsc.VectorSubcoreMesh(core_axis_name="c", subcore_axis_name="s")


def _bvec(v16, j):
    return jnp.take(v16, jnp.full((16,), j, dtype=jnp.int32))


@functools.partial(
    pl.kernel,
    out_type=jax.ShapeDtypeStruct((_NC, _NPAD, _EMB), jnp.float32),
    mesh=_mesh,
    scratch_types=dict(
        cb0=pltpu.VMEM((_K,), jnp.int32),
        cb1=pltpu.VMEM((_K,), jnp.int32),
        cb2=pltpu.VMEM((_K,), jnp.int32),
        vb0=pltpu.VMEM((_K,), jnp.float32),
        vb1=pltpu.VMEM((_K,), jnp.float32),
        vb2=pltpu.VMEM((_K,), jnp.float32),
        rb0=pltpu.VMEM((_K,), jnp.int32),
        rb1=pltpu.VMEM((_K,), jnp.int32),
        rb2=pltpu.VMEM((_K,), jnp.int32),
        gb0=pltpu.VMEM((_K, _EMB), jnp.float32),
        gb1=pltpu.VMEM((_K, _EMB), jnp.float32),
        gb2=pltpu.VMEM((_K, _EMB), jnp.float32),
        semc0=pltpu.SemaphoreType.DMA, semc1=pltpu.SemaphoreType.DMA,
        semc2=pltpu.SemaphoreType.DMA, semg0=pltpu.SemaphoreType.DMA,
        semg1=pltpu.SemaphoreType.DMA, semg2=pltpu.SemaphoreType.DMA,
        sems0=pltpu.SemaphoreType.DMA, sems1=pltpu.SemaphoreType.DMA,
        sems2=pltpu.SemaphoreType.DMA, semr0=pltpu.SemaphoreType.DMA,
        semr1=pltpu.SemaphoreType.DMA, semr2=pltpu.SemaphoreType.DMA,
        acc=pltpu.VMEM_SHARED((_NPAD, _EMB), jnp.float32),
    ),
)
def _layer(ego, colsr, valsr, rowsr, part, *, cb0, cb1, cb2, vb0, vb1, vb2,
           rb0, rb1, rb2, gb0, gb1, gb2, semc0, semc1, semc2, semg0, semg1,
           semg2, sems0, sems1, sems2, semr0, semr1, semr2, acc):
    cid = lax.axis_index("c")
    sid = lax.axis_index("s")
    wid = cid * _NS + sid
    cb = (cb0, cb1, cb2)
    vb = (vb0, vb1, vb2)
    rb = (rb0, rb1, rb2)
    gb = (gb0, gb1, gb2)
    semc = (semc0, semc1, semc2)
    semg = (semg0, semg1, semg2)
    sems = (sems0, sems1, sems2)
    semr = (semr0, semr1, semr2)
    nch = jnp.where(cid == 0, _CH0, _CH1)
    cbase = jnp.where(cid == 0, sid * _CH0, 16 * _CH0 + sid * _CH1)

    z16 = jnp.zeros((16,), jnp.float32)

    def _zero_row(i, carry):
        for cc in range(_EMB // 16):
            gb0[i, pl.ds(cc * 16, 16)] = z16
        return carry

    lax.fori_loop(0, _K, _zero_row, 0)
    rstart = sid * _RPT
    for z in range(_RPT // _K):
        pltpu.sync_copy(gb0, acc.at[pl.ds(rstart + z * _K, _K)])
    pltpu.sync_copy(gb0.at[pl.ds(0, _RPT % _K)],
                    acc.at[pl.ds(rstart + (_RPT // _K) * _K, _RPT % _K)])

    plsc.subcore_barrier()

    def _meta_issue(t, r):
        pltpu.async_copy(colsr.at[cbase + t], cb[r], semc[r])
        pltpu.async_copy(valsr.at[cbase + t], vb[r], semc[r])

    def _meta_wait(t, r):
        pltpu.make_async_copy(colsr.at[cbase + t], cb[r], semc[r]).wait()
        pltpu.make_async_copy(valsr.at[cbase + t], vb[r], semc[r]).wait()

    def _rows_issue(t, r):
        pltpu.async_copy(rowsr.at[cbase + t], rb[r], semr[r])

    def _rows_wait(t, r):
        pltpu.make_async_copy(rowsr.at[cbase + t], rb[r], semr[r]).wait()

    def _gather_issue(t, r):
        pltpu.async_copy(ego.at[cb[r]], gb[r], semg[r])

    def _gather_wait(r):
        pltpu.make_async_copy(ego.at[cb[r]], gb[r], semg[r]).wait()

    def _scatter_issue(t, r):
        pltpu.async_copy(gb[r], acc.at[rb[r]], sems[r], add=True)

    def _scatter_wait(r):
        pltpu.make_async_copy(gb[r], acc.at[rb[r]], sems[r]).wait()

    def _compute(r):
        gbr = gb[r]
        vbr = vb[r]

        def _grp(g, carry):
            v16 = vbr[pl.ds(g * 16, 16)]
            for j in range(16):
                e = g * 16 + j
                bv = _bvec(v16, j)
                for cc in range(_EMB // 16):
                    gbr[e, pl.ds(cc * 16, 16)] = (
                        gbr[e, pl.ds(cc * 16, 16)] * bv)
            return carry

        lax.fori_loop(0, _K // 16, _grp, 0)

    def _slot(t, r, do_next, do_meta, do_free):
        rn = (r + 1) % 3
        if do_next:
            _meta_wait(t + 1, rn)
            if do_free:
                _scatter_wait(rn)
            _rows_issue(t + 1, rn)
            _gather_issue(t + 1, rn)
        _gather_wait(r)
        _compute(r)
        _rows_wait(t, r)
        _scatter_issue(t, r)
        if do_meta:
            _meta_issue(t + 2, (r + 2) % 3)

    _meta_issue(0, 0)
    _meta_issue(1, 1)
    _rows_issue(0, 0)
    _meta_wait(0, 0)
    _gather_issue(0, 0)

    _slot(0, 0, True, True, False)
    _slot(1, 1, True, True, False)
    _slot(2, 2, True, True, True)

    def _main(q, carry):
        t = 3 + q * 3
        _slot(t, 0, True, True, True)
        _slot(t + 1, 1, True, True, True)
        _slot(t + 2, 2, True, True, True)
        return carry

    lax.fori_loop(0, (nch - 6) // 3, _main, 0)

    _slot(nch - 3, 0, True, True, True)
    _slot(nch - 2, 1, True, False, True)
    _slot(nch - 1, 2, False, False, False)

    _scatter_wait(0)
    _scatter_wait(1)
    _scatter_wait(2)

    plsc.subcore_barrier()

    pltpu.sync_copy(acc.at[pl.ds(rstart, _RPT)],
                    part.at[cid, pl.ds(rstart, _RPT)])


_CBLK = 632


def _combine_body(p_ref, o_ref):
    o_ref[...] = p_ref[0] + p_ref[1]


def _combine(part):
    return pl.pallas_call(
        _combine_body,
        out_shape=jax.ShapeDtypeStruct((_NPAD, _EMB), jnp.float32),
        grid=(_NPAD // _CBLK,),
        in_specs=[pl.BlockSpec((_NC, _CBLK, _EMB), lambda i: (0, i, 0))],
        out_specs=pl.BlockSpec((_CBLK, _EMB), lambda i: (i, 0)),
    )(part)


_B2 = 2 * _BATCH
_GK = 32
_GCH = _B2 // _GK // _NW


@functools.partial(
    pl.kernel,
    out_type=jax.ShapeDtypeStruct((_B2, _EMB), jnp.float32),
    mesh=_mesh,
    scratch_types=dict(
        ibuf=pltpu.VMEM((_GCH, _GK), jnp.int32),
        gbuf=pltpu.VMEM((_GK, _EMB), jnp.float32),
        obuf=pltpu.VMEM((_GK, _EMB), jnp.float32),
        sem=pltpu.SemaphoreType.DMA,
    ),
)
def _gather_mean(ego0, ego1, ego2, p3a, p3b, bidx, out, *, ibuf, gbuf, obuf,
                 sem):
    cid = lax.axis_index("c")
    sid = lax.axis_index("s")
    wid = cid * _NS + sid
    pltpu.sync_copy(bidx.at[wid], ibuf)
    quarter = jnp.full((16,), 0.25, dtype=jnp.float32)

    def _chunk(t, carry):
        def _acc_rows(first):
            def _row(i, c2):
                for cc in range(_EMB // 16):
                    g = gbuf[i, pl.ds(cc * 16, 16)]
                    if first:
                        obuf[i, pl.ds(cc * 16, 16)] = g
                    else:
                        obuf[i, pl.ds(cc * 16, 16)] = (
                            obuf[i, pl.ds(cc * 16, 16)] + g)
                return c2
            lax.fori_loop(0, _GK, _row, 0)

        for s, src in enumerate((ego0, ego1, ego2, p3a, p3b)):
            pltpu.async_copy(src.at[ibuf.at[t]], gbuf, sem).wait()
            _acc_rows(s == 0)

        def _scale(i, c2):
            for cc in range(_EMB // 16):
                obuf[i, pl.ds(cc * 16, 16)] = (
                    obuf[i, pl.ds(cc * 16, 16)] * quarter)
            return c2

        lax.fori_loop(0, _GK, _scale, 0)
        pltpu.sync_copy(obuf, out.at[pl.ds((wid * _GCH + t) * _GK, _GK)])
        return carry

    lax.fori_loop(0, _GCH, _chunk, 0)


def kernel(users, items, user_emb, item_emb, adj_rows, adj_cols, adj_vals):
    ego0 = jnp.concatenate([user_emb, item_emb], axis=0)

    npad = _EPAD - _E
    cols_p = jnp.concatenate([adj_cols, jnp.zeros((npad,), jnp.int32)])
    rows_p = jnp.concatenate([adj_rows, jnp.zeros((npad,), jnp.int32)])
    vals_p = jnp.concatenate([adj_vals, jnp.zeros((npad,), jnp.float32)])
    colsr = cols_p.reshape(_NW * _CH, _K)
    valsr = vals_p.reshape(_NW * _CH, _K)
    rowsr = rows_p.reshape(_NW * _CH, _K)

    p1 = _layer(ego0, colsr, valsr, rowsr)
    ego1 = _combine(p1)
    p2 = _layer(ego1, colsr, valsr, rowsr)
    ego2 = _combine(p2)
    p3 = _layer(ego2, colsr, valsr, rowsr)

    bidx = jnp.concatenate([users, items + _USER]).reshape(_NW, _GCH, _GK)
    out = _gather_mean(ego0, ego1, ego2, p3[0], p3[1], bidx)
    return out[:_BATCH], out[_BATCH:]


# speedup vs baseline: 1.1543x; 1.0385x over previous
import functools

import jax
import jax.numpy as jnp
from jax import lax
from jax.experimental import pallas as pl
from jax.experimental.pallas import tpu as pltpu
from jax.experimental.pallas import tpu_sc as plsc

_USER = 5000
_N = 10000
_NPAD = 10112
_EMB = 128
_E = 320000
_BATCH = 4096
_NC, _NS = 2, 16
_NW = _NC * _NS
_K = 96
_CH = 105
_CH0 = 132
_CH1 = 2 * _CH - _CH0
_EPT = _CH * _K
_EPAD = _EPT * _NW
_RPT = _NPAD // _NS

_mesh = plsc.VectorSubcoreMesh(core_axis_name="c", subcore_axis_name="s")


def _bvec(v16, j):
    return jnp.take(v16, jnp.full((16,), j, dtype=jnp.int32))


@functools.partial(
    pl.kernel,
    out_type=jax.ShapeDtypeStruct((_NC, _NPAD, _EMB), jnp.float32),
    mesh=_mesh,
    scratch_types=dict(
        cb0=pltpu.VMEM((_K,), jnp.int32),
        cb1=pltpu.VMEM((_K,), jnp.int32),
        cb2=pltpu.VMEM((_K,), jnp.int32),
        vb0=pltpu.VMEM((_K,), jnp.float32),
        vb1=pltpu.VMEM((_K,), jnp.float32),
        vb2=pltpu.VMEM((_K,), jnp.float32),
        rb0=pltpu.VMEM((_K,), jnp.int32),
        rb1=pltpu.VMEM((_K,), jnp.int32),
        rb2=pltpu.VMEM((_K,), jnp.int32),
        gb0=pltpu.VMEM((_K, _EMB), jnp.float32),
        gb1=pltpu.VMEM((_K, _EMB), jnp.float32),
        gb2=pltpu.VMEM((_K, _EMB), jnp.float32),
        semc0=pltpu.SemaphoreType.DMA, semc1=pltpu.SemaphoreType.DMA,
        semc2=pltpu.SemaphoreType.DMA, semg0=pltpu.SemaphoreType.DMA,
        semg1=pltpu.SemaphoreType.DMA, semg2=pltpu.SemaphoreType.DMA,
        sems0=pltpu.SemaphoreType.DMA, sems1=pltpu.SemaphoreType.DMA,
        sems2=pltpu.SemaphoreType.DMA, semr0=pltpu.SemaphoreType.DMA,
        semr1=pltpu.SemaphoreType.DMA, semr2=pltpu.SemaphoreType.DMA,
        acc=pltpu.VMEM_SHARED((_NPAD, _EMB), jnp.float32),
    ),
)
def _layer(ego, colsr, valsr, rowsr, part, *, cb0, cb1, cb2, vb0, vb1, vb2,
           rb0, rb1, rb2, gb0, gb1, gb2, semc0, semc1, semc2, semg0, semg1,
           semg2, sems0, sems1, sems2, semr0, semr1, semr2, acc):
    cid = lax.axis_index("c")
    sid = lax.axis_index("s")
    wid = cid * _NS + sid
    cb = (cb0, cb1, cb2)
    vb = (vb0, vb1, vb2)
    rb = (rb0, rb1, rb2)
    gb = (gb0, gb1, gb2)
    semc = (semc0, semc1, semc2)
    semg = (semg0, semg1, semg2)
    sems = (sems0, sems1, sems2)
    semr = (semr0, semr1, semr2)
    nch = jnp.where(cid == 0, _CH0, _CH1)
    cbase = jnp.where(cid == 0, sid * _CH0, 16 * _CH0 + sid * _CH1)

    z16 = jnp.zeros((16,), jnp.float32)

    def _zero_row(i, carry):
        for cc in range(_EMB // 16):
            gb0[i, pl.ds(cc * 16, 16)] = z16
        return carry

    lax.fori_loop(0, _K, _zero_row, 0)
    rstart = sid * _RPT
    for z in range(_RPT // _K):
        pltpu.sync_copy(gb0, acc.at[pl.ds(rstart + z * _K, _K)])
    pltpu.sync_copy(gb0.at[pl.ds(0, _RPT % _K)],
                    acc.at[pl.ds(rstart + (_RPT // _K) * _K, _RPT % _K)])

    plsc.subcore_barrier()

    def _meta_issue(t, r):
        pltpu.async_copy(colsr.at[cbase + t], cb[r], semc[r])
        pltpu.async_copy(valsr.at[cbase + t], vb[r], semc[r])

    def _meta_wait(t, r):
        pltpu.make_async_copy(colsr.at[cbase + t], cb[r], semc[r]).wait()
        pltpu.make_async_copy(valsr.at[cbase + t], vb[r], semc[r]).wait()

    def _rows_issue(t, r):
        pltpu.async_copy(rowsr.at[cbase + t], rb[r], semr[r])

    def _rows_wait(t, r):
        pltpu.make_async_copy(rowsr.at[cbase + t], rb[r], semr[r]).wait()

    def _gather_issue(t, r):
        pltpu.async_copy(ego.at[cb[r]], gb[r], semg[r])

    def _gather_wait(r):
        pltpu.make_async_copy(ego.at[cb[r]], gb[r], semg[r]).wait()

    def _scatter_issue(t, r):
        pltpu.async_copy(gb[r], acc.at[rb[r]], sems[r], add=True)

    def _scatter_wait(r):
        pltpu.make_async_copy(gb[r], acc.at[rb[r]], sems[r]).wait()

    def _compute(r):
        gbr = gb[r]
        vbr = vb[r]

        def _grp(g, carry):
            v16 = vbr[pl.ds(g * 16, 16)]
            for j in range(16):
                e = g * 16 + j
                bv = _bvec(v16, j)
                for cc in range(_EMB // 16):
                    gbr[e, pl.ds(cc * 16, 16)] = (
                        gbr[e, pl.ds(cc * 16, 16)] * bv)
            return carry

        lax.fori_loop(0, _K // 16, _grp, 0)

    def _slot(t, r, do_next, do_meta, do_free):
        rn = (r + 1) % 3
        if do_next:
            _meta_wait(t + 1, rn)
            if do_free:
                _scatter_wait(rn)
            _rows_issue(t + 1, rn)
            _gather_issue(t + 1, rn)
        _gather_wait(r)
        _compute(r)
        _rows_wait(t, r)
        _scatter_issue(t, r)
        if do_meta:
            _meta_issue(t + 2, (r + 2) % 3)

    _meta_issue(0, 0)
    _meta_issue(1, 1)
    _rows_issue(0, 0)
    _meta_wait(0, 0)
    _gather_issue(0, 0)

    _slot(0, 0, True, True, False)
    _slot(1, 1, True, True, False)
    _slot(2, 2, True, True, True)

    def _main(q, carry):
        t = 3 + q * 3
        _slot(t, 0, True, True, True)
        _slot(t + 1, 1, True, True, True)
        _slot(t + 2, 2, True, True, True)
        return carry

    lax.fori_loop(0, (nch - 6) // 3, _main, 0)

    _slot(nch - 3, 0, True, True, True)
    _slot(nch - 2, 1, True, False, True)
    _slot(nch - 1, 2, False, False, False)

    _scatter_wait(0)
    _scatter_wait(1)
    _scatter_wait(2)

    plsc.subcore_barrier()

    pltpu.sync_copy(acc.at[pl.ds(rstart, _RPT)],
                    part.at[cid, pl.ds(rstart, _RPT)])


_CBLK = 632


def _combine_body(p_ref, o_ref):
    o_ref[...] = p_ref[0] + p_ref[1]


def _combine(part):
    return pl.pallas_call(
        _combine_body,
        out_shape=jax.ShapeDtypeStruct((_NPAD, _EMB), jnp.float32),
        grid=(_NPAD // _CBLK,),
        in_specs=[pl.BlockSpec((_NC, _CBLK, _EMB), lambda i: (0, i, 0))],
        out_specs=pl.BlockSpec((_CBLK, _EMB), lambda i: (i, 0)),
    )(part)


_B2 = 2 * _BATCH
_GK = 32
_GCH = _B2 // _GK // _NW


@functools.partial(
    pl.kernel,
    out_type=jax.ShapeDtypeStruct((_B2, _EMB), jnp.float32),
    mesh=_mesh,
    scratch_types=dict(
        ibuf=pltpu.VMEM((_GCH, _GK), jnp.int32),
        gbuf=pltpu.VMEM((_GK, _EMB), jnp.float32),
        obuf=pltpu.VMEM((_GK, _EMB), jnp.float32),
        sem=pltpu.SemaphoreType.DMA,
    ),
)
def _gather_mean(ego0, ego1, ego2, p3a, p3b, bidx, out, *, ibuf, gbuf, obuf,
                 sem):
    cid = lax.axis_index("c")
    sid = lax.axis_index("s")
    wid = cid * _NS + sid
    pltpu.sync_copy(bidx.at[wid], ibuf)
    quarter = jnp.full((16,), 0.25, dtype=jnp.float32)

    def _chunk(t, carry):
        def _acc_rows(first):
            def _row(i, c2):
                for cc in range(_EMB // 16):
                    g = gbuf[i, pl.ds(cc * 16, 16)]
                    if first:
                        obuf[i, pl.ds(cc * 16, 16)] = g
                    else:
                        obuf[i, pl.ds(cc * 16, 16)] = (
                            obuf[i, pl.ds(cc * 16, 16)] + g)
                return c2
            lax.fori_loop(0, _GK, _row, 0)

        for s, src in enumerate((ego0, ego1, ego2, p3a, p3b)):
            pltpu.async_copy(src.at[ibuf.at[t]], gbuf, sem).wait()
            _acc_rows(s == 0)

        def _scale(i, c2):
            for cc in range(_EMB // 16):
                obuf[i, pl.ds(cc * 16, 16)] = (
                    obuf[i, pl.ds(cc * 16, 16)] * quarter)
            return c2

        lax.fori_loop(0, _GK, _scale, 0)
        pltpu.sync_copy(obuf, out.at[pl.ds((wid * _GCH + t) * _GK, _GK)])
        return carry

    lax.fori_loop(0, _GCH, _chunk, 0)


def kernel(users, items, user_emb, item_emb, adj_rows, adj_cols, adj_vals):
    ego0 = jnp.concatenate([user_emb, item_emb], axis=0)

    npad = _EPAD - _E
    cols_p = jnp.concatenate([adj_cols, jnp.zeros((npad,), jnp.int32)])
    rows_p = jnp.concatenate([adj_rows, jnp.zeros((npad,), jnp.int32)])
    vals_p = jnp.concatenate([adj_vals, jnp.zeros((npad,), jnp.float32)])
    colsr = cols_p.reshape(_NW * _CH, _K)
    valsr = vals_p.reshape(_NW * _CH, _K)
    rowsr = rows_p.reshape(_NW * _CH, _K)

    p1 = _layer(ego0, colsr, valsr, rowsr)
    ego1 = _combine(p1)
    p2 = _layer(ego1, colsr, valsr, rowsr)
    ego2 = _combine(p2)
    p3 = _layer(ego2, colsr, valsr, rowsr)

    bidx = jnp.concatenate([users, items + _USER]).reshape(_NW, _GCH, _GK)
    out = _gather_mean(ego0, ego1, ego2, p3[0], p3[1], bidx)
    return out[:_BATCH], out[_BATCH:]


# speedup vs baseline: 1.1812x; 1.0233x over previous
import functools

import jax
import jax.numpy as jnp
from jax import lax
from jax.experimental import pallas as pl
from jax.experimental.pallas import tpu as pltpu
from jax.experimental.pallas import tpu_sc as plsc

_USER = 5000
_N = 10000
_NPAD = 10112
_EMB = 128
_E = 320000
_BATCH = 4096
_NC, _NS = 2, 16
_NW = _NC * _NS
_K = 96
_CH = 105
_CH0 = 138
_CH1 = 2 * _CH - _CH0
_EPT = _CH * _K
_EPAD = _EPT * _NW
_RPT = _NPAD // _NS

_mesh = plsc.VectorSubcoreMesh(core_axis_name="c", subcore_axis_name="s")


def _bvec(v16, j):
    return jnp.take(v16, jnp.full((16,), j, dtype=jnp.int32))


@functools.partial(
    pl.kernel,
    out_type=jax.ShapeDtypeStruct((_NC, _NPAD, _EMB), jnp.float32),
    mesh=_mesh,
    scratch_types=dict(
        cb0=pltpu.VMEM((_K,), jnp.int32),
        cb1=pltpu.VMEM((_K,), jnp.int32),
        cb2=pltpu.VMEM((_K,), jnp.int32),
        vb0=pltpu.VMEM((_K,), jnp.float32),
        vb1=pltpu.VMEM((_K,), jnp.float32),
        vb2=pltpu.VMEM((_K,), jnp.float32),
        rb0=pltpu.VMEM((_K,), jnp.int32),
        rb1=pltpu.VMEM((_K,), jnp.int32),
        rb2=pltpu.VMEM((_K,), jnp.int32),
        gb0=pltpu.VMEM((_K, _EMB), jnp.float32),
        gb1=pltpu.VMEM((_K, _EMB), jnp.float32),
        gb2=pltpu.VMEM((_K, _EMB), jnp.float32),
        semc0=pltpu.SemaphoreType.DMA, semc1=pltpu.SemaphoreType.DMA,
        semc2=pltpu.SemaphoreType.DMA, semg0=pltpu.SemaphoreType.DMA,
        semg1=pltpu.SemaphoreType.DMA, semg2=pltpu.SemaphoreType.DMA,
        sems0=pltpu.SemaphoreType.DMA, sems1=pltpu.SemaphoreType.DMA,
        sems2=pltpu.SemaphoreType.DMA, semr0=pltpu.SemaphoreType.DMA,
        semr1=pltpu.SemaphoreType.DMA, semr2=pltpu.SemaphoreType.DMA,
        acc=pltpu.VMEM_SHARED((_NPAD, _EMB), jnp.float32),
    ),
)
def _layer(ego, colsr, valsr, rowsr, part, *, cb0, cb1, cb2, vb0, vb1, vb2,
           rb0, rb1, rb2, gb0, gb1, gb2, semc0, semc1, semc2, semg0, semg1,
           semg2, sems0, sems1, sems2, semr0, semr1, semr2, acc):
    cid = lax.axis_index("c")
    sid = lax.axis_index("s")
    wid = cid * _NS + sid
    cb = (cb0, cb1, cb2)
    vb = (vb0, vb1, vb2)
    rb = (rb0, rb1, rb2)
    gb = (gb0, gb1, gb2)
    semc = (semc0, semc1, semc2)
    semg = (semg0, semg1, semg2)
    sems = (sems0, sems1, sems2)
    semr = (semr0, semr1, semr2)
    nch = jnp.where(cid == 0, _CH0, _CH1)
    cbase = jnp.where(cid == 0, sid * _CH0, 16 * _CH0 + sid * _CH1)

    z16 = jnp.zeros((16,), jnp.float32)

    def _zero_row(i, carry):
        for cc in range(_EMB // 16):
            gb0[i, pl.ds(cc * 16, 16)] = z16
        return carry

    lax.fori_loop(0, _K, _zero_row, 0)
    rstart = sid * _RPT
    for z in range(_RPT // _K):
        pltpu.sync_copy(gb0, acc.at[pl.ds(rstart + z * _K, _K)])
    pltpu.sync_copy(gb0.at[pl.ds(0, _RPT % _K)],
                    acc.at[pl.ds(rstart + (_RPT // _K) * _K, _RPT % _K)])

    plsc.subcore_barrier()

    def _meta_issue(t, r):
        pltpu.async_copy(colsr.at[cbase + t], cb[r], semc[r])
        pltpu.async_copy(valsr.at[cbase + t], vb[r], semc[r])

    def _meta_wait(t, r):
        pltpu.make_async_copy(colsr.at[cbase + t], cb[r], semc[r]).wait()
        pltpu.make_async_copy(valsr.at[cbase + t], vb[r], semc[r]).wait()

    def _rows_issue(t, r):
        pltpu.async_copy(rowsr.at[cbase + t], rb[r], semr[r])

    def _rows_wait(t, r):
        pltpu.make_async_copy(rowsr.at[cbase + t], rb[r], semr[r]).wait()

    def _gather_issue(t, r):
        pltpu.async_copy(ego.at[cb[r]], gb[r], semg[r])

    def _gather_wait(r):
        pltpu.make_async_copy(ego.at[cb[r]], gb[r], semg[r]).wait()

    def _scatter_issue(t, r):
        pltpu.async_copy(gb[r], acc.at[rb[r]], sems[r], add=True)

    def _scatter_wait(r):
        pltpu.make_async_copy(gb[r], acc.at[rb[r]], sems[r]).wait()

    def _compute(r):
        gbr = gb[r]
        vbr = vb[r]

        def _grp(g, carry):
            v16 = vbr[pl.ds(g * 16, 16)]
            for j in range(16):
                e = g * 16 + j
                bv = _bvec(v16, j)
                for cc in range(_EMB // 16):
                    gbr[e, pl.ds(cc * 16, 16)] = (
                        gbr[e, pl.ds(cc * 16, 16)] * bv)
            return carry

        lax.fori_loop(0, _K // 16, _grp, 0)

    def _slot(t, r, do_next, do_meta, do_free):
        rn = (r + 1) % 3
        if do_next:
            _meta_wait(t + 1, rn)
            if do_free:
                _scatter_wait(rn)
            _rows_issue(t + 1, rn)
            _gather_issue(t + 1, rn)
        _gather_wait(r)
        _compute(r)
        _rows_wait(t, r)
        _scatter_issue(t, r)
        if do_meta:
            _meta_issue(t + 2, (r + 2) % 3)

    _meta_issue(0, 0)
    _meta_issue(1, 1)
    _rows_issue(0, 0)
    _meta_wait(0, 0)
    _gather_issue(0, 0)

    _slot(0, 0, True, True, False)
    _slot(1, 1, True, True, False)
    _slot(2, 2, True, True, True)

    def _main(q, carry):
        t = 3 + q * 3
        _slot(t, 0, True, True, True)
        _slot(t + 1, 1, True, True, True)
        _slot(t + 2, 2, True, True, True)
        return carry

    lax.fori_loop(0, (nch - 6) // 3, _main, 0)

    _slot(nch - 3, 0, True, True, True)
    _slot(nch - 2, 1, True, False, True)
    _slot(nch - 1, 2, False, False, False)

    _scatter_wait(0)
    _scatter_wait(1)
    _scatter_wait(2)

    plsc.subcore_barrier()

    pltpu.sync_copy(acc.at[pl.ds(rstart, _RPT)],
                    part.at[cid, pl.ds(rstart, _RPT)])


_CBLK = 632


def _combine_body(p_ref, o_ref):
    o_ref[...] = p_ref[0] + p_ref[1]


def _combine(part):
    return pl.pallas_call(
        _combine_body,
        out_shape=jax.ShapeDtypeStruct((_NPAD, _EMB), jnp.float32),
        grid=(_NPAD // _CBLK,),
        in_specs=[pl.BlockSpec((_NC, _CBLK, _EMB), lambda i: (0, i, 0))],
        out_specs=pl.BlockSpec((_CBLK, _EMB), lambda i: (i, 0)),
    )(part)


_B2 = 2 * _BATCH
_GK = 32
_GCH = _B2 // _GK // _NW


@functools.partial(
    pl.kernel,
    out_type=jax.ShapeDtypeStruct((_B2, _EMB), jnp.float32),
    mesh=_mesh,
    scratch_types=dict(
        ibuf=pltpu.VMEM((_GCH, _GK), jnp.int32),
        gbuf=pltpu.VMEM((_GK, _EMB), jnp.float32),
        obuf=pltpu.VMEM((_GK, _EMB), jnp.float32),
        sem=pltpu.SemaphoreType.DMA,
    ),
)
def _gather_mean(ego0, ego1, ego2, p3a, p3b, bidx, out, *, ibuf, gbuf, obuf,
                 sem):
    cid = lax.axis_index("c")
    sid = lax.axis_index("s")
    wid = cid * _NS + sid
    pltpu.sync_copy(bidx.at[wid], ibuf)
    quarter = jnp.full((16,), 0.25, dtype=jnp.float32)

    def _chunk(t, carry):
        def _acc_rows(first):
            def _row(i, c2):
                for cc in range(_EMB // 16):
                    g = gbuf[i, pl.ds(cc * 16, 16)]
                    if first:
                        obuf[i, pl.ds(cc * 16, 16)] = g
                    else:
                        obuf[i, pl.ds(cc * 16, 16)] = (
                            obuf[i, pl.ds(cc * 16, 16)] + g)
                return c2
            lax.fori_loop(0, _GK, _row, 0)

        for s, src in enumerate((ego0, ego1, ego2, p3a, p3b)):
            pltpu.async_copy(src.at[ibuf.at[t]], gbuf, sem).wait()
            _acc_rows(s == 0)

        def _scale(i, c2):
            for cc in range(_EMB // 16):
                obuf[i, pl.ds(cc * 16, 16)] = (
                    obuf[i, pl.ds(cc * 16, 16)] * quarter)
            return c2

        lax.fori_loop(0, _GK, _scale, 0)
        pltpu.sync_copy(obuf, out.at[pl.ds((wid * _GCH + t) * _GK, _GK)])
        return carry

    lax.fori_loop(0, _GCH, _chunk, 0)


def kernel(users, items, user_emb, item_emb, adj_rows, adj_cols, adj_vals):
    ego0 = jnp.concatenate([user_emb, item_emb], axis=0)

    npad = _EPAD - _E
    cols_p = jnp.concatenate([adj_cols, jnp.zeros((npad,), jnp.int32)])
    rows_p = jnp.concatenate([adj_rows, jnp.zeros((npad,), jnp.int32)])
    vals_p = jnp.concatenate([adj_vals, jnp.zeros((npad,), jnp.float32)])
    colsr = cols_p.reshape(_NW * _CH, _K)
    valsr = vals_p.reshape(_NW * _CH, _K)
    rowsr = rows_p.reshape(_NW * _CH, _K)

    p1 = _layer(ego0, colsr, valsr, rowsr)
    ego1 = _combine(p1)
    p2 = _layer(ego1, colsr, valsr, rowsr)
    ego2 = _combine(p2)
    p3 = _layer(ego2, colsr, valsr, rowsr)

    bidx = jnp.concatenate([users, items + _USER]).reshape(_NW, _GCH, _GK)
    out = _gather_mean(ego0, ego1, ego2, p3[0], p3[1], bidx)
    return out[:_BATCH], out[_BATCH:]


# speedup vs baseline: 1.1964x; 1.0128x over previous
import functools

import jax
import jax.numpy as jnp
from jax import lax
from jax.experimental import pallas as pl
from jax.experimental.pallas import tpu as pltpu
from jax.experimental.pallas import tpu_sc as plsc

_USER = 5000
_N = 10000
_NPAD = 10112
_EMB = 128
_E = 320000
_BATCH = 4096
_NC, _NS = 2, 16
_NW = _NC * _NS
_K = 96
_CH = 105
_CH0 = 150
_CH1 = 2 * _CH - _CH0
_EPT = _CH * _K
_EPAD = _EPT * _NW
_RPT = _NPAD // _NS

_mesh = plsc.VectorSubcoreMesh(core_axis_name="c", subcore_axis_name="s")


def _bvec(v16, j):
    return jnp.take(v16, jnp.full((16,), j, dtype=jnp.int32))


@functools.partial(
    pl.kernel,
    out_type=jax.ShapeDtypeStruct((_NC, _NPAD, _EMB), jnp.float32),
    mesh=_mesh,
    scratch_types=dict(
        cb0=pltpu.VMEM((_K,), jnp.int32),
        cb1=pltpu.VMEM((_K,), jnp.int32),
        cb2=pltpu.VMEM((_K,), jnp.int32),
        vb0=pltpu.VMEM((_K,), jnp.float32),
        vb1=pltpu.VMEM((_K,), jnp.float32),
        vb2=pltpu.VMEM((_K,), jnp.float32),
        rb0=pltpu.VMEM((_K,), jnp.int32),
        rb1=pltpu.VMEM((_K,), jnp.int32),
        rb2=pltpu.VMEM((_K,), jnp.int32),
        gb0=pltpu.VMEM((_K, _EMB), jnp.float32),
        gb1=pltpu.VMEM((_K, _EMB), jnp.float32),
        gb2=pltpu.VMEM((_K, _EMB), jnp.float32),
        semc0=pltpu.SemaphoreType.DMA, semc1=pltpu.SemaphoreType.DMA,
        semc2=pltpu.SemaphoreType.DMA, semg0=pltpu.SemaphoreType.DMA,
        semg1=pltpu.SemaphoreType.DMA, semg2=pltpu.SemaphoreType.DMA,
        sems0=pltpu.SemaphoreType.DMA, sems1=pltpu.SemaphoreType.DMA,
        sems2=pltpu.SemaphoreType.DMA, semr0=pltpu.SemaphoreType.DMA,
        semr1=pltpu.SemaphoreType.DMA, semr2=pltpu.SemaphoreType.DMA,
        acc=pltpu.VMEM_SHARED((_NPAD, _EMB), jnp.float32),
    ),
)
def _layer(ego, colsr, valsr, rowsr, part, *, cb0, cb1, cb2, vb0, vb1, vb2,
           rb0, rb1, rb2, gb0, gb1, gb2, semc0, semc1, semc2, semg0, semg1,
           semg2, sems0, sems1, sems2, semr0, semr1, semr2, acc):
    cid = lax.axis_index("c")
    sid = lax.axis_index("s")
    wid = cid * _NS + sid
    cb = (cb0, cb1, cb2)
    vb = (vb0, vb1, vb2)
    rb = (rb0, rb1, rb2)
    gb = (gb0, gb1, gb2)
    semc = (semc0, semc1, semc2)
    semg = (semg0, semg1, semg2)
    sems = (sems0, sems1, sems2)
    semr = (semr0, semr1, semr2)
    nch = jnp.where(cid == 0, _CH0, _CH1)
    cbase = jnp.where(cid == 0, sid * _CH0, 16 * _CH0 + sid * _CH1)

    z16 = jnp.zeros((16,), jnp.float32)

    def _zero_row(i, carry):
        for cc in range(_EMB // 16):
            gb0[i, pl.ds(cc * 16, 16)] = z16
        return carry

    lax.fori_loop(0, _K, _zero_row, 0)
    rstart = sid * _RPT
    for z in range(_RPT // _K):
        pltpu.sync_copy(gb0, acc.at[pl.ds(rstart + z * _K, _K)])
    pltpu.sync_copy(gb0.at[pl.ds(0, _RPT % _K)],
                    acc.at[pl.ds(rstart + (_RPT // _K) * _K, _RPT % _K)])

    plsc.subcore_barrier()

    def _meta_issue(t, r):
        pltpu.async_copy(colsr.at[cbase + t], cb[r], semc[r])
        pltpu.async_copy(valsr.at[cbase + t], vb[r], semc[r])

    def _meta_wait(t, r):
        pltpu.make_async_copy(colsr.at[cbase + t], cb[r], semc[r]).wait()
        pltpu.make_async_copy(valsr.at[cbase + t], vb[r], semc[r]).wait()

    def _rows_issue(t, r):
        pltpu.async_copy(rowsr.at[cbase + t], rb[r], semr[r])

    def _rows_wait(t, r):
        pltpu.make_async_copy(rowsr.at[cbase + t], rb[r], semr[r]).wait()

    def _gather_issue(t, r):
        pltpu.async_copy(ego.at[cb[r]], gb[r], semg[r])

    def _gather_wait(r):
        pltpu.make_async_copy(ego.at[cb[r]], gb[r], semg[r]).wait()

    def _scatter_issue(t, r):
        pltpu.async_copy(gb[r], acc.at[rb[r]], sems[r], add=True)

    def _scatter_wait(r):
        pltpu.make_async_copy(gb[r], acc.at[rb[r]], sems[r]).wait()

    def _compute(r):
        gbr = gb[r]
        vbr = vb[r]

        def _grp(g, carry):
            v16 = vbr[pl.ds(g * 16, 16)]
            for j in range(16):
                e = g * 16 + j
                bv = _bvec(v16, j)
                for cc in range(_EMB // 16):
                    gbr[e, pl.ds(cc * 16, 16)] = (
                        gbr[e, pl.ds(cc * 16, 16)] * bv)
            return carry

        lax.fori_loop(0, _K // 16, _grp, 0)

    def _slot(t, r, do_next, do_meta, do_free):
        rn = (r + 1) % 3
        if do_next:
            _meta_wait(t + 1, rn)
            if do_free:
                _scatter_wait(rn)
            _rows_issue(t + 1, rn)
            _gather_issue(t + 1, rn)
        _gather_wait(r)
        _compute(r)
        _rows_wait(t, r)
        _scatter_issue(t, r)
        if do_meta:
            _meta_issue(t + 2, (r + 2) % 3)

    _meta_issue(0, 0)
    _meta_issue(1, 1)
    _rows_issue(0, 0)
    _meta_wait(0, 0)
    _gather_issue(0, 0)

    _slot(0, 0, True, True, False)
    _slot(1, 1, True, True, False)
    _slot(2, 2, True, True, True)

    def _main(q, carry):
        t = 3 + q * 3
        _slot(t, 0, True, True, True)
        _slot(t + 1, 1, True, True, True)
        _slot(t + 2, 2, True, True, True)
        return carry

    lax.fori_loop(0, (nch - 6) // 3, _main, 0)

    _slot(nch - 3, 0, True, True, True)
    _slot(nch - 2, 1, True, False, True)
    _slot(nch - 1, 2, False, False, False)

    _scatter_wait(0)
    _scatter_wait(1)
    _scatter_wait(2)

    plsc.subcore_barrier()

    pltpu.sync_copy(acc.at[pl.ds(rstart, _RPT)],
                    part.at[cid, pl.ds(rstart, _RPT)])


_CBLK = 632


def _combine_body(p_ref, o_ref):
    o_ref[...] = p_ref[0] + p_ref[1]


def _combine(part):
    return pl.pallas_call(
        _combine_body,
        out_shape=jax.ShapeDtypeStruct((_NPAD, _EMB), jnp.float32),
        grid=(_NPAD // _CBLK,),
        in_specs=[pl.BlockSpec((_NC, _CBLK, _EMB), lambda i: (0, i, 0))],
        out_specs=pl.BlockSpec((_CBLK, _EMB), lambda i: (i, 0)),
    )(part)


_B2 = 2 * _BATCH
_GK = 32
_GCH = _B2 // _GK // _NW


@functools.partial(
    pl.kernel,
    out_type=jax.ShapeDtypeStruct((_B2, _EMB), jnp.float32),
    mesh=_mesh,
    scratch_types=dict(
        ibuf=pltpu.VMEM((_GCH, _GK), jnp.int32),
        gbuf=pltpu.VMEM((_GK, _EMB), jnp.float32),
        obuf=pltpu.VMEM((_GK, _EMB), jnp.float32),
        sem=pltpu.SemaphoreType.DMA,
    ),
)
def _gather_mean(ego0, ego1, ego2, p3a, p3b, bidx, out, *, ibuf, gbuf, obuf,
                 sem):
    cid = lax.axis_index("c")
    sid = lax.axis_index("s")
    wid = cid * _NS + sid
    pltpu.sync_copy(bidx.at[wid], ibuf)
    quarter = jnp.full((16,), 0.25, dtype=jnp.float32)

    def _chunk(t, carry):
        def _acc_rows(first):
            def _row(i, c2):
                for cc in range(_EMB // 16):
                    g = gbuf[i, pl.ds(cc * 16, 16)]
                    if first:
                        obuf[i, pl.ds(cc * 16, 16)] = g
                    else:
                        obuf[i, pl.ds(cc * 16, 16)] = (
                            obuf[i, pl.ds(cc * 16, 16)] + g)
                return c2
            lax.fori_loop(0, _GK, _row, 0)

        for s, src in enumerate((ego0, ego1, ego2, p3a, p3b)):
            pltpu.async_copy(src.at[ibuf.at[t]], gbuf, sem).wait()
            _acc_rows(s == 0)

        def _scale(i, c2):
            for cc in range(_EMB // 16):
                obuf[i, pl.ds(cc * 16, 16)] = (
                    obuf[i, pl.ds(cc * 16, 16)] * quarter)
            return c2

        lax.fori_loop(0, _GK, _scale, 0)
        pltpu.sync_copy(obuf, out.at[pl.ds((wid * _GCH + t) * _GK, _GK)])
        return carry

    lax.fori_loop(0, _GCH, _chunk, 0)


def kernel(users, items, user_emb, item_emb, adj_rows, adj_cols, adj_vals):
    ego0 = jnp.concatenate([user_emb, item_emb], axis=0)

    npad = _EPAD - _E
    cols_p = jnp.concatenate([adj_cols, jnp.zeros((npad,), jnp.int32)])
    rows_p = jnp.concatenate([adj_rows, jnp.zeros((npad,), jnp.int32)])
    vals_p = jnp.concatenate([adj_vals, jnp.zeros((npad,), jnp.float32)])
    colsr = cols_p.reshape(_NW * _CH, _K)
    valsr = vals_p.reshape(_NW * _CH, _K)
    rowsr = rows_p.reshape(_NW * _CH, _K)

    p1 = _layer(ego0, colsr, valsr, rowsr)
    ego1 = _combine(p1)
    p2 = _layer(ego1, colsr, valsr, rowsr)
    ego2 = _combine(p2)
    p3 = _layer(ego2, colsr, valsr, rowsr)

    bidx = jnp.concatenate([users, items + _USER]).reshape(_NW, _GCH, _GK)
    out = _gather_mean(ego0, ego1, ego2, p3[0], p3[1], bidx)
    return out[:_BATCH], out[_BATCH:]
